# initial kernel scaffold (unmeasured)
import jax
import jax.numpy as jnp
from jax import lax
from jax.experimental import pallas as pl
from jax.experimental.pallas import tpu as pltpu

N_DEV = 32
M_PER = 128


def kernel(x, w_mat):
    m_glob, k_per = x.shape
    k_glob, n = w_mat.shape

    def body(x_ref, w_ref, out_ref, gathered_ref, send_sems, recv_sems):
        me = lax.axis_index("i")

        gathered_ref[:, pl.ds(me * M_PER, M_PER)] = x_ref[pl.ds(me * M_PER, M_PER), :]

        sends = []
        for off in range(1, N_DEV):
            dst = (me + off) % N_DEV
            rdma = pltpu.make_async_remote_copy(
                src_ref=x_ref.at[pl.ds(dst * M_PER, M_PER), :],
                dst_ref=gathered_ref.at[:, pl.ds(me * M_PER, M_PER)],
                send_sem=send_sems.at[off],
                recv_sem=recv_sems.at[me],
                device_id=(dst,),
                device_id_type=pl.DeviceIdType.MESH,
            )
            rdma.start()
            sends.append(rdma)

        for off in range(1, N_DEV):
            src = (me + off) % N_DEV
            recv = pltpu.make_async_remote_copy(
                src_ref=x_ref.at[pl.ds(src * M_PER, M_PER), :],
                dst_ref=gathered_ref.at[:, pl.ds(src * M_PER, M_PER)],
                send_sem=send_sems.at[off],
                recv_sem=recv_sems.at[src],
                device_id=(src,),
                device_id_type=pl.DeviceIdType.MESH,
            )
            recv.wait_recv()

        out_ref[...] = jnp.dot(
            gathered_ref[...], w_ref[...], preferred_element_type=jnp.float32
        )

        for rdma in sends:
            rdma.wait_send()

    return pl.pallas_call(
        body,
        out_shape=jax.ShapeDtypeStruct((M_PER, n), jnp.float32),
        in_specs=[
            pl.BlockSpec(memory_space=pltpu.VMEM),
            pl.BlockSpec(memory_space=pltpu.VMEM),
        ],
        out_specs=pl.BlockSpec(memory_space=pltpu.VMEM),
        scratch_shapes=[
            pltpu.VMEM((M_PER, k_glob), jnp.float32),
            pltpu.SemaphoreType.DMA((N_DEV,)),
            pltpu.SemaphoreType.DMA((N_DEV,)),
        ],
        compiler_params=pltpu.CompilerParams(
            collective_id=0,
            vmem_limit_bytes=100 * 1024 * 1024,
        ),
    )(x, w_mat)


# baseline (device time: 53757 ns/iter reference)
import jax
import jax.numpy as jnp
from jax import lax
from jax.experimental import pallas as pl
from jax.experimental.pallas import tpu as pltpu

N_DEV = 32
M_PER = 128


def kernel(x, w_mat):
    m_glob, k_per = x.shape
    k_glob, n = w_mat.shape

    def body(x_ref, w_ref, out_ref, gathered_ref, send_sems, recv_sems):
        me = lax.axis_index("i")

        gathered_ref[:, pl.ds(me * M_PER, M_PER)] = x_ref[pl.ds(me * M_PER, M_PER), :]

        sends = []
        for off in range(1, N_DEV):
            dst = (me + off) % N_DEV
            rdma = pltpu.make_async_remote_copy(
                src_ref=x_ref.at[pl.ds(dst * M_PER, M_PER), :],
                dst_ref=gathered_ref.at[:, pl.ds(me * M_PER, M_PER)],
                send_sem=send_sems.at[off],
                recv_sem=recv_sems.at[me],
                device_id=(dst,),
                device_id_type=pl.DeviceIdType.MESH,
            )
            rdma.start()
            sends.append(rdma)

        for off in range(1, N_DEV):
            src = (me + off) % N_DEV
            recv = pltpu.make_async_remote_copy(
                src_ref=x_ref.at[pl.ds(src * M_PER, M_PER), :],
                dst_ref=gathered_ref.at[:, pl.ds(src * M_PER, M_PER)],
                send_sem=send_sems.at[off],
                recv_sem=recv_sems.at[src],
                device_id=(src,),
                device_id_type=pl.DeviceIdType.MESH,
            )
            recv.wait_recv()

        out_ref[...] = jnp.dot(
            gathered_ref[...], w_ref[...], preferred_element_type=jnp.float32
        )

        for rdma in sends:
            rdma.wait_send()

    return pl.pallas_call(
        body,
        out_shape=jax.ShapeDtypeStruct((M_PER, n), jnp.float32),
        in_specs=[
            pl.BlockSpec(memory_space=pltpu.VMEM),
            pl.BlockSpec(memory_space=pltpu.VMEM),
        ],
        out_specs=pl.BlockSpec(memory_space=pltpu.VMEM),
        scratch_shapes=[
            pltpu.VMEM((M_PER, k_glob), jnp.float32),
            pltpu.SemaphoreType.DMA((N_DEV,)),
            pltpu.SemaphoreType.DMA((N_DEV,)),
        ],
        compiler_params=pltpu.CompilerParams(
            vmem_limit_bytes=100 * 1024 * 1024,
        ),
    )(x, w_mat)
